# baseline (speedup 1.0000x reference)
import jax
import jax.numpy as jnp
from jax.experimental import pallas as pl
from jax.experimental.pallas import tpu as pltpu


def _pick_bm(n, cap):
    for bm in (cap, 200, 80, 40, 16, 8):
        if bm <= cap and n % bm == 0 and bm % 8 == 0:
            return bm
    return n


def _h_kernel(x_ref, w0_ref, adj_ref, h_ref, adjbf_ref, xw0_ref):
    @pl.when(pl.program_id(0) == 0)
    def _():
        xw0_ref[...] = jnp.dot(
            x_ref[...], w0_ref[...], preferred_element_type=jnp.float32
        ).astype(jnp.bfloat16)

    adj_bf = adj_ref[...].astype(jnp.bfloat16)
    adjbf_ref[...] = adj_bf
    h_ref[...] = jnp.tanh(
        jnp.dot(adj_bf, xw0_ref[...], preferred_element_type=jnp.float32)
    )


def _head_kernel(h_ref, wcat_ref, adjbf_ref, eps_ref, z_ref, m_ref, s_ref, hw_ref):
    latent = m_ref.shape[1]

    @pl.when(pl.program_id(0) == 0)
    def _():
        hw_ref[...] = jnp.dot(
            h_ref[...], wcat_ref[...], preferred_element_type=jnp.float32
        ).astype(jnp.bfloat16)

    acc = jnp.dot(
        adjbf_ref[...], hw_ref[...], preferred_element_type=jnp.float32
    )
    m = acc[:, :latent]
    s = jnp.maximum(acc[:, latent:], 0.0) + 0.0001
    m_ref[...] = m
    s_ref[...] = s
    z_ref[...] = eps_ref[...] * s + m


def kernel(adj, x, W0, Wm, Ws):
    n, d_in = x.shape
    hidden = W0.shape[1]
    latent = Wm.shape[1]

    bm1 = _pick_bm(n, 200)
    h, adj_bf = pl.pallas_call(
        _h_kernel,
        grid=(n // bm1,),
        in_specs=[
            pl.BlockSpec((n, d_in), lambda i: (0, 0)),
            pl.BlockSpec((d_in, hidden), lambda i: (0, 0)),
            pl.BlockSpec((bm1, n), lambda i: (i, 0)),
        ],
        out_specs=[
            pl.BlockSpec((bm1, hidden), lambda i: (i, 0)),
            pl.BlockSpec((bm1, n), lambda i: (i, 0)),
        ],
        out_shape=[
            jax.ShapeDtypeStruct((n, hidden), jnp.float32),
            jax.ShapeDtypeStruct((n, n), jnp.bfloat16),
        ],
        scratch_shapes=[pltpu.VMEM((n, hidden), jnp.bfloat16)],
        compiler_params=pltpu.CompilerParams(
            dimension_semantics=("arbitrary",),
        ),
    )(x, W0, adj)

    wcat = jnp.concatenate([Wm, Ws], axis=1)
    eps = jax.random.normal(jax.random.key(42), (n, latent), dtype=jnp.float32)

    bm2 = _pick_bm(n, 400)
    out_sds = jax.ShapeDtypeStruct((n, latent), jnp.float32)
    lat_spec = pl.BlockSpec((bm2, latent), lambda i: (i, 0))
    z, m_q_z, std_q_z = pl.pallas_call(
        _head_kernel,
        grid=(n // bm2,),
        in_specs=[
            pl.BlockSpec((n, hidden), lambda i: (0, 0)),
            pl.BlockSpec((hidden, 2 * latent), lambda i: (0, 0)),
            pl.BlockSpec((bm2, n), lambda i: (i, 0)),
            lat_spec,
        ],
        out_specs=[lat_spec, lat_spec, lat_spec],
        out_shape=[out_sds, out_sds, out_sds],
        scratch_shapes=[pltpu.VMEM((n, 2 * latent), jnp.bfloat16)],
        compiler_params=pltpu.CompilerParams(
            dimension_semantics=("arbitrary",),
        ),
    )(h, wcat, adj_bf, eps)

    return (z, m_q_z, std_q_z)


# speedup vs baseline: 1.0739x; 1.0739x over previous
import functools

import jax
import jax.numpy as jnp
from jax.experimental import pallas as pl
from jax.experimental.pallas import tpu as pltpu


def _pick_bm(n, cap=400):
    for bm in (cap, 200, 80, 40, 16, 8):
        if bm <= cap and n % bm == 0 and bm % 8 == 0:
            return bm
    return n


def _fused_kernel(
    nb, bm,
    x_ref, w0_ref, wcat_ref, adj_ref, eps_ref,
    z_ref, m_ref, s_ref,
    xw0_ref, h_ref, hw_ref,
):
    latent = m_ref.shape[1]
    i = pl.program_id(0)

    @pl.when(i == 0)
    def _():
        xw0_ref[...] = jnp.dot(
            x_ref[...], w0_ref[...], preferred_element_type=jnp.float32
        ).astype(jnp.bfloat16)

    @pl.when(i < nb)
    def _():
        h_ref[pl.ds(i * bm, bm), :] = jnp.tanh(
            jnp.dot(
                adj_ref[...].astype(jnp.bfloat16),
                xw0_ref[...],
                preferred_element_type=jnp.float32,
            )
        ).astype(jnp.bfloat16)

    @pl.when(i == nb)
    def _():
        hw_ref[...] = jnp.dot(
            h_ref[...], wcat_ref[...], preferred_element_type=jnp.float32
        ).astype(jnp.bfloat16)

    @pl.when(i >= nb)
    def _():
        acc = jnp.dot(
            adj_ref[...].astype(jnp.bfloat16),
            hw_ref[...],
            preferred_element_type=jnp.float32,
        )
        m = acc[:, :latent]
        s = jnp.maximum(acc[:, latent:], 0.0) + 0.0001
        m_ref[...] = m
        s_ref[...] = s
        z_ref[...] = eps_ref[...] * s + m


def kernel(adj, x, W0, Wm, Ws):
    n, d_in = x.shape
    hidden = W0.shape[1]
    latent = Wm.shape[1]
    bm = _pick_bm(n)
    nb = n // bm

    wcat = jnp.concatenate([Wm, Ws], axis=1)
    eps = jax.random.normal(jax.random.key(42), (n, latent), dtype=jnp.float32)

    out_sds = jax.ShapeDtypeStruct((n, latent), jnp.float32)
    ph2_spec = pl.BlockSpec(
        (bm, latent), lambda i: (jnp.maximum(i - nb, 0), 0)
    )
    z, m_q_z, std_q_z = pl.pallas_call(
        functools.partial(_fused_kernel, nb, bm),
        grid=(2 * nb,),
        in_specs=[
            pl.BlockSpec((n, d_in), lambda i: (0, 0)),
            pl.BlockSpec((d_in, hidden), lambda i: (0, 0)),
            pl.BlockSpec((hidden, 2 * latent), lambda i: (0, 0)),
            pl.BlockSpec((bm, n), lambda i: (jax.lax.rem(i, nb), 0)),
            ph2_spec,
        ],
        out_specs=[ph2_spec, ph2_spec, ph2_spec],
        out_shape=[out_sds, out_sds, out_sds],
        scratch_shapes=[
            pltpu.VMEM((n, hidden), jnp.bfloat16),
            pltpu.VMEM((n, hidden), jnp.bfloat16),
            pltpu.VMEM((n, 2 * latent), jnp.bfloat16),
        ],
        compiler_params=pltpu.CompilerParams(
            dimension_semantics=("arbitrary",),
        ),
    )(x, W0, wcat, adj, eps)

    return (z, m_q_z, std_q_z)
